# Initial kernel scaffold; baseline (speedup 1.0000x reference)
import functools

import jax
import jax.numpy as jnp
from jax import lax
from jax.experimental import pallas as pl
from jax.experimental.pallas import tpu as pltpu
from jax.experimental.pallas import tpu_sc as plsc

N = 10000
R2 = 10000
D = 128
E = 160000
T = 3
P = 2 * E
TBL = N * R2

NC = 2
NS = 16
NW = NC * NS

NPAD = 10240
DUMMY = 10000
RPT = NPAD // NS

CA = 80
PWA = P // NW
NCHA = PWA // CA

CB = 128
EPAD = 1280 * CB
NCHB = EPAD // (NW * CB)

TCB = 500


@functools.lru_cache(maxsize=None)
def _mesh():
    return plsc.VectorSubcoreMesh(core_axis_name="c", subcore_axis_name="s")


def _wid():
    return lax.axis_index("c") * NS + lax.axis_index("s")


def _zero_fill(zbuf):
    rows = zbuf.shape[0]

    @pl.loop(0, rows)
    def _(i):
        for k in range(D // 16):
            zbuf[i, pl.ds(k * 16, 16)] = jnp.zeros((16,), jnp.float32)


def _zero_fill_1d(zvec):
    n = zvec.shape[0]

    @pl.loop(0, n // 16)
    def _(i):
        zvec[pl.ds(i * 16, 16)] = jnp.zeros((16,), jnp.float32)


@functools.lru_cache(maxsize=None)
def _make_a1():
    @functools.partial(
        pl.kernel,
        out_type=jax.ShapeDtypeStruct((TBL,), jnp.int32),
        mesh=_mesh(),
        scratch_types=[
            pltpu.VMEM((2, CA), jnp.int32),
            pltpu.VMEM((2, CA), jnp.int32),
            pltpu.VMEM((2, CA), jnp.int32),
            pltpu.VMEM((2, CA), jnp.int32),
            pltpu.SemaphoreType.DMA((2,)),
            pltpu.SemaphoreType.DMA((2,)),
        ],
    )
    def a1(ents, rels, table, entb, relb, pidb, valb, semi, sems):
        base = _wid() * PWA

        @pl.loop(0, NCHA)
        def _(j):
            off = base + j * CA
            pltpu.async_copy(ents.at[pl.ds(off, CA)], entb.at[0], semi.at[0])
            pltpu.async_copy(rels.at[pl.ds(off, CA)], relb.at[0], semi.at[1])
            pltpu.make_async_copy(ents.at[pl.ds(off, CA)], entb.at[0], semi.at[0]).wait()
            pltpu.make_async_copy(rels.at[pl.ds(off, CA)], relb.at[0], semi.at[1]).wait()
            for m in range(CA // 16):
                sl = pl.ds(m * 16, 16)
                pidb[0, sl] = entb[0, sl] * R2 + relb[0, sl]
                valb[0, sl] = (off + m * 16) + lax.iota(jnp.int32, 16)
            pltpu.async_copy(valb.at[0], table.at[pidb.at[0]], sems.at[0]).wait()

    return a1


@functools.lru_cache(maxsize=None)
def _make_a2():
    @functools.partial(
        pl.kernel,
        out_type=(
            jax.ShapeDtypeStruct((NC, NPAD, D), jnp.float32),
            jax.ShapeDtypeStruct((NC, NPAD), jnp.float32),
        ),
        mesh=_mesh(),
        scratch_types=[
            pltpu.VMEM((2, CA), jnp.int32),
            pltpu.VMEM((2, CA), jnp.int32),
            pltpu.VMEM((2, CA), jnp.int32),
            pltpu.VMEM((2, CA), jnp.int32),
            pltpu.VMEM((2, CA), jnp.int32),
            pltpu.VMEM((2, CA, D), jnp.float32),
            pltpu.VMEM((1, CA), jnp.float32),
            pltpu.VMEM((128, D), jnp.float32),
            pltpu.VMEM((RPT,), jnp.float32),
            pltpu.VMEM_SHARED((NPAD, D), jnp.float32),
            pltpu.VMEM_SHARED((NPAD,), jnp.float32),
            pltpu.SemaphoreType.DMA((2,)),
            pltpu.SemaphoreType.DMA((2,)),
        ],
    )
    def a2(ents, rels, table, h, sums_out, cnt_out,
           entb, relb, pidb, tvb, selb, rowsb, onesb, zbuf, zvec,
           sums_sh, cnt_sh, semi, semg):
        cid = lax.axis_index("c")
        sid = lax.axis_index("s")
        base = _wid() * PWA
        r0 = sid * RPT

        _zero_fill(zbuf)
        _zero_fill_1d(zvec)
        for m in range(CA // 16):
            onesb[0, pl.ds(m * 16, 16)] = jnp.ones((16,), jnp.float32)

        @pl.loop(0, RPT // 128)
        def _(jj):
            pltpu.sync_copy(zbuf, sums_sh.at[pl.ds(r0 + jj * 128, 128)])

        pltpu.sync_copy(zvec, cnt_sh.at[pl.ds(r0, RPT)])
        plsc.subcore_barrier()

        @pl.loop(0, NCHA)
        def _(j):
            off = base + j * CA
            pltpu.async_copy(ents.at[pl.ds(off, CA)], entb.at[0], semi.at[0])
            pltpu.async_copy(rels.at[pl.ds(off, CA)], relb.at[0], semi.at[1])
            pltpu.make_async_copy(ents.at[pl.ds(off, CA)], entb.at[0], semi.at[0]).wait()
            pltpu.make_async_copy(rels.at[pl.ds(off, CA)], relb.at[0], semi.at[1]).wait()
            for m in range(CA // 16):
                sl = pl.ds(m * 16, 16)
                pidb[0, sl] = entb[0, sl] * R2 + relb[0, sl]
            g1 = pltpu.async_copy(table.at[pidb.at[0]], tvb.at[0], semg.at[0])
            g2 = pltpu.async_copy(h.at[entb.at[0]], rowsb.at[0], semg.at[1])
            g1.wait()
            g2.wait()
            for m in range(CA // 16):
                sl = pl.ds(m * 16, 16)
                val16 = (off + m * 16) + lax.iota(jnp.int32, 16)
                first = tvb[0, sl] == val16
                selb[0, sl] = jnp.where(first, relb[0, sl], DUMMY)
            pltpu.sync_copy(rowsb.at[0], sums_sh.at[selb.at[0]], add=True)
            pltpu.sync_copy(onesb.at[0], cnt_sh.at[selb.at[0]], add=True)

        plsc.subcore_barrier()

        @pl.loop(0, RPT // 128)
        def _(jj):
            pltpu.sync_copy(sums_sh.at[pl.ds(r0 + jj * 128, 128)],
                            sums_out.at[cid, pl.ds(r0 + jj * 128, 128)])

        pltpu.sync_copy(cnt_sh.at[pl.ds(r0, RPT)], cnt_out.at[cid, pl.ds(r0, RPT)])

    return a2


@functools.lru_cache(maxsize=None)
def _make_b(with_deg):
    outs = [jax.ShapeDtypeStruct((NC, NPAD, D), jnp.float32)]
    scratch = [
        pltpu.VMEM((2, CB), jnp.int32),
        pltpu.VMEM((2, CB), jnp.int32),
        pltpu.VMEM((2, CB), jnp.int32),
        pltpu.VMEM((2, CB, D), jnp.float32),
        pltpu.VMEM((2, CB, D), jnp.float32),
        pltpu.VMEM((1, CB), jnp.float32),
        pltpu.VMEM((128, D), jnp.float32),
        pltpu.VMEM((RPT,), jnp.float32),
        pltpu.VMEM_SHARED((NPAD, D), jnp.float32),
        pltpu.VMEM_SHARED((NPAD,), jnp.float32),
        pltpu.SemaphoreType.DMA((3,)),
        pltpu.SemaphoreType.DMA((2,)),
    ]
    if with_deg:
        outs.append(jax.ShapeDtypeStruct((NC, NPAD), jnp.float32))

    @functools.partial(
        pl.kernel,
        out_type=tuple(outs),
        mesh=_mesh(),
        scratch_types=scratch,
    )
    def b(src, rel, dst, taba, tabb, *args):
        if with_deg:
            (agg_out, deg_out, sb, rb, db, rowsa, rowsb, onesb, zbuf, zvec,
             agg_sh, deg_sh, semi, semg) = args
        else:
            (agg_out, sb, rb, db, rowsa, rowsb, onesb, zbuf, zvec,
             agg_sh, deg_sh, semi, semg) = args
            deg_out = None
        cid = lax.axis_index("c")
        sid = lax.axis_index("s")
        wid = _wid()
        r0 = sid * RPT

        _zero_fill(zbuf)
        _zero_fill_1d(zvec)
        for m in range(CB // 16):
            onesb[0, pl.ds(m * 16, 16)] = jnp.ones((16,), jnp.float32)

        @pl.loop(0, RPT // 128)
        def _(jj):
            pltpu.sync_copy(zbuf, agg_sh.at[pl.ds(r0 + jj * 128, 128)])

        if with_deg:
            pltpu.sync_copy(zvec, deg_sh.at[pl.ds(r0, RPT)])
        plsc.subcore_barrier()

        @pl.loop(0, NCHB)
        def _(k):
            off = (wid * NCHB + k) * CB
            pltpu.async_copy(src.at[pl.ds(off, CB)], sb.at[0], semi.at[0])
            pltpu.async_copy(rel.at[pl.ds(off, CB)], rb.at[0], semi.at[1])
            pltpu.async_copy(dst.at[pl.ds(off, CB)], db.at[0], semi.at[2])
            pltpu.make_async_copy(src.at[pl.ds(off, CB)], sb.at[0], semi.at[0]).wait()
            pltpu.make_async_copy(rel.at[pl.ds(off, CB)], rb.at[0], semi.at[1]).wait()
            pltpu.make_async_copy(dst.at[pl.ds(off, CB)], db.at[0], semi.at[2]).wait()
            g1 = pltpu.async_copy(taba.at[sb.at[0]], rowsa.at[0], semg.at[0])
            g2 = pltpu.async_copy(tabb.at[rb.at[0]], rowsb.at[0], semg.at[1])
            g1.wait()
            g2.wait()
            pltpu.sync_copy(rowsa.at[0], agg_sh.at[db.at[0]], add=True)
            pltpu.sync_copy(rowsb.at[0], agg_sh.at[db.at[0]], add=True)
            if with_deg:
                pltpu.sync_copy(onesb.at[0], deg_sh.at[db.at[0]], add=True)

        plsc.subcore_barrier()

        @pl.loop(0, RPT // 128)
        def _(jj):
            pltpu.sync_copy(agg_sh.at[pl.ds(r0 + jj * 128, 128)],
                            agg_out.at[cid, pl.ds(r0 + jj * 128, 128)])

        if with_deg:
            pltpu.sync_copy(deg_sh.at[pl.ds(r0, RPT)], deg_out.at[cid, pl.ds(r0, RPT)])

    return b


def _tc_norm(x):
    def body(x_ref, o_ref):
        v = x_ref[...]
        n = jnp.sqrt(jnp.sum(v * v, axis=1, keepdims=True))
        o_ref[...] = v / jnp.maximum(n, 1e-12)

    return pl.pallas_call(
        body,
        out_shape=jax.ShapeDtypeStruct((N, D), jnp.float32),
        grid=(N // TCB,),
        in_specs=[pl.BlockSpec((TCB, D), lambda i: (i, 0))],
        out_specs=pl.BlockSpec((TCB, D), lambda i: (i, 0)),
    )(x)


def _tc_rel(sums2, counts3, srel, hidden, wihs, wihd, whh, bih, bhh):
    def body(s_ref, c_ref, sr_ref, hid_ref, wihs_ref, wihd_ref, whh_ref,
             bih_ref, bhh_ref, o_ref):
        s = s_ref[0] + s_ref[1]
        c = c_ref[0] + c_ref[1]
        dyn = jnp.where(c > 0.0, s / jnp.maximum(c, 1.0), 0.0)
        gi = (jnp.dot(sr_ref[...], wihs_ref[...], preferred_element_type=jnp.float32)
              + jnp.dot(dyn, wihd_ref[...], preferred_element_type=jnp.float32)
              + bih_ref[...])
        gh = (jnp.dot(hid_ref[...], whh_ref[...], preferred_element_type=jnp.float32)
              + bhh_ref[...])
        rg = jax.nn.sigmoid(gi[:, :D] + gh[:, :D])
        zg = jax.nn.sigmoid(gi[:, D:2 * D] + gh[:, D:2 * D])
        ng = jnp.tanh(gi[:, 2 * D:] + rg * gh[:, 2 * D:])
        o_ref[...] = (1.0 - zg) * ng + zg * hid_ref[...]

    return pl.pallas_call(
        body,
        out_shape=jax.ShapeDtypeStruct((R2, D), jnp.float32),
        grid=(R2 // TCB,),
        in_specs=[
            pl.BlockSpec((NC, TCB, D), lambda i: (0, i, 0)),
            pl.BlockSpec((NC, TCB, 1), lambda i: (0, i, 0)),
            pl.BlockSpec((TCB, D), lambda i: (i, 0)),
            pl.BlockSpec((TCB, D), lambda i: (i, 0)),
            pl.BlockSpec((D, 3 * D), lambda i: (0, 0)),
            pl.BlockSpec((D, 3 * D), lambda i: (0, 0)),
            pl.BlockSpec((D, 3 * D), lambda i: (0, 0)),
            pl.BlockSpec((1, 3 * D), lambda i: (0, 0)),
            pl.BlockSpec((1, 3 * D), lambda i: (0, 0)),
        ],
        out_specs=pl.BlockSpec((TCB, D), lambda i: (i, 0)),
    )(sums2, counts3, srel, hidden, wihs, wihd, whh, bih, bhh)


def _tc_layer0(agg2, deg3, cur, nw, lw):
    def body(a_ref, d_ref, cur_ref, nw_ref, lw_ref, o_ref):
        a = a_ref[0] + a_ref[1]
        d = jnp.maximum(d_ref[0] + d_ref[1], 1.0)
        o_ref[...] = (jnp.dot(a, nw_ref[...], preferred_element_type=jnp.float32) / d
                      + jnp.dot(cur_ref[...], lw_ref[...],
                                preferred_element_type=jnp.float32))

    return pl.pallas_call(
        body,
        out_shape=jax.ShapeDtypeStruct((N, D), jnp.float32),
        grid=(N // TCB,),
        in_specs=[
            pl.BlockSpec((NC, TCB, D), lambda i: (0, i, 0)),
            pl.BlockSpec((NC, TCB, 1), lambda i: (0, i, 0)),
            pl.BlockSpec((TCB, D), lambda i: (i, 0)),
            pl.BlockSpec((D, D), lambda i: (0, 0)),
            pl.BlockSpec((D, D), lambda i: (0, 0)),
        ],
        out_specs=pl.BlockSpec((TCB, D), lambda i: (i, 0)),
    )(agg2, deg3, cur, nw, lw)


def _tc_layer1(agg2, deg3, cur, h, nw, lw, gw, gb):
    def body(a_ref, d_ref, cur_ref, h_ref, nw_ref, lw_ref, gw_ref, gb_ref, o_ref):
        a = a_ref[0] + a_ref[1]
        d = jnp.maximum(d_ref[0] + d_ref[1], 1.0)
        cur2 = (jnp.dot(a, nw_ref[...], preferred_element_type=jnp.float32) / d
                + jnp.dot(cur_ref[...], lw_ref[...],
                          preferred_element_type=jnp.float32))
        g = jax.nn.sigmoid(
            jnp.dot(h_ref[...], gw_ref[...], preferred_element_type=jnp.float32)
            + gb_ref[...])
        o_ref[...] = g * cur2 + (1.0 - g) * h_ref[...]

    return pl.pallas_call(
        body,
        out_shape=jax.ShapeDtypeStruct((N, D), jnp.float32),
        grid=(N // TCB,),
        in_specs=[
            pl.BlockSpec((NC, TCB, D), lambda i: (0, i, 0)),
            pl.BlockSpec((NC, TCB, 1), lambda i: (0, i, 0)),
            pl.BlockSpec((TCB, D), lambda i: (i, 0)),
            pl.BlockSpec((TCB, D), lambda i: (i, 0)),
            pl.BlockSpec((D, D), lambda i: (0, 0)),
            pl.BlockSpec((D, D), lambda i: (0, 0)),
            pl.BlockSpec((D, D), lambda i: (0, 0)),
            pl.BlockSpec((1, D), lambda i: (0, 0)),
        ],
        out_specs=pl.BlockSpec((TCB, D), lambda i: (i, 0)),
    )(agg2, deg3, cur, h, nw, lw, gw, gb)


def kernel(edges, static_entity_embed, static_relation_embed, gate_weight,
           gate_bias, gru_w_ih, gru_w_hh, gru_b_ih, gru_b_hh, neigh_w, loop_w):
    et = edges.transpose(0, 2, 1)
    wihs = gru_w_ih[:, :D].T
    wihd = gru_w_ih[:, D:].T
    whh = gru_w_hh.T
    bih = gru_b_ih.reshape(1, 3 * D)
    bhh = gru_b_hh.reshape(1, 3 * D)
    gb = gate_bias.reshape(1, D)

    pad0 = jnp.zeros((EPAD - E,), jnp.int32)
    padd = jnp.full((EPAD - E,), DUMMY, jnp.int32)

    a1 = _make_a1()
    a2 = _make_a2()
    b_deg = _make_b(True)
    b_nodeg = _make_b(False)

    h = _tc_norm(static_entity_embed)
    evolved = static_relation_embed
    outs = []
    for t in range(T):
        src, rel, dst = et[t, 0], et[t, 1], et[t, 2]
        ents = jnp.concatenate([src, dst])
        rels2 = jnp.concatenate([rel, rel])
        src_p = jnp.concatenate([src, pad0])
        rel_p = jnp.concatenate([rel, pad0])
        dst_p = jnp.concatenate([dst, padd])

        table = a1(ents, rels2)
        sums2, counts2 = a2(ents, rels2, table, h)
        evolved = _tc_rel(sums2, counts2.reshape(NC, NPAD, 1),
                          static_relation_embed, evolved,
                          wihs, wihd, whh, bih, bhh)

        agg2, deg2 = b_deg(src_p, rel_p, dst_p, h, evolved)
        deg3 = deg2.reshape(NC, NPAD, 1)
        cur1 = _tc_layer0(agg2, deg3, h, neigh_w[0], loop_w[0])

        agg2b = b_nodeg(src_p, rel_p, dst_p, cur1, evolved)
        h = _tc_layer1(agg2b, deg3, cur1, h, neigh_w[1], loop_w[1],
                       gate_weight, gb)
        outs.append(h)

    return jnp.stack(outs, axis=0), evolved


# speedup vs baseline: 2.8858x; 2.8858x over previous
import functools

import jax
import jax.numpy as jnp
from jax import lax
from jax.experimental import pallas as pl
from jax.experimental.pallas import tpu as pltpu
from jax.experimental.pallas import tpu_sc as plsc

N = 10000
R2 = 10000
D = 128
E = 160000
T = 3
P = 2 * E
TBL = N * R2

NC = 2
NS = 16
NW = NC * NS

NPAD = 10240
DUMMY = 10000
RPT = NPAD // NS

CA = 80
PWA = P // NW
NCHA = PWA // CA

CB = 64
NCHB = 80
EPAD = NW * NCHB * CB

ZR = 64

TCB = 1000


@functools.lru_cache(maxsize=None)
def _mesh():
    return plsc.VectorSubcoreMesh(core_axis_name="c", subcore_axis_name="s")


def _wid():
    return lax.axis_index("c") * NS + lax.axis_index("s")


def _zero_fill(zbuf):
    rows = zbuf.shape[0]

    @pl.loop(0, rows)
    def _(i):
        for k in range(D // 16):
            zbuf[i, pl.ds(k * 16, 16)] = jnp.zeros((16,), jnp.float32)


def _zero_fill_1d(zvec):
    n = zvec.shape[0]

    @pl.loop(0, n // 16)
    def _(i):
        zvec[pl.ds(i * 16, 16)] = jnp.zeros((16,), jnp.float32)


@functools.lru_cache(maxsize=None)
def _make_a1():
    @functools.partial(
        pl.kernel,
        out_type=pltpu.HBM((TBL,), jnp.int32),
        mesh=_mesh(),
        scratch_types=[
            pltpu.VMEM((2, CA), jnp.int32),
            pltpu.VMEM((2, CA), jnp.int32),
            pltpu.VMEM((2, CA), jnp.int32),
            pltpu.VMEM((2, CA), jnp.int32),
            pltpu.SemaphoreType.DMA((2,)),
            pltpu.SemaphoreType.DMA((2,)),
        ],
    )
    def a1(ents, rels, table, entb, relb, pidb, valb, semi, sems):
        base = _wid() * PWA

        @pl.loop(0, NCHA)
        def _(j):
            off = base + j * CA
            pltpu.async_copy(ents.at[pl.ds(off, CA)], entb.at[0], semi.at[0])
            pltpu.async_copy(rels.at[pl.ds(off, CA)], relb.at[0], semi.at[1])
            pltpu.make_async_copy(ents.at[pl.ds(off, CA)], entb.at[0], semi.at[0]).wait()
            pltpu.make_async_copy(rels.at[pl.ds(off, CA)], relb.at[0], semi.at[1]).wait()
            for m in range(CA // 16):
                sl = pl.ds(m * 16, 16)
                pidb[0, sl] = entb[0, sl] * R2 + relb[0, sl]
                valb[0, sl] = (off + m * 16) + lax.iota(jnp.int32, 16)
            pltpu.async_copy(valb.at[0], table.at[pidb.at[0]], sems.at[0]).wait()

    return a1


@functools.lru_cache(maxsize=None)
def _make_a2():
    @functools.partial(
        pl.kernel,
        out_type=(
            pltpu.HBM((NC, NPAD, D), jnp.float32),
            pltpu.HBM((NC, NPAD), jnp.float32),
        ),
        mesh=_mesh(),
        scratch_types=[
            pltpu.VMEM((2, CA), jnp.int32),
            pltpu.VMEM((2, CA), jnp.int32),
            pltpu.VMEM((2, CA), jnp.int32),
            pltpu.VMEM((2, CA), jnp.int32),
            pltpu.VMEM((2, CA), jnp.int32),
            pltpu.VMEM((2, CA, D), jnp.float32),
            pltpu.VMEM((1, CA), jnp.float32),
            pltpu.VMEM((ZR, D), jnp.float32),
            pltpu.VMEM((RPT,), jnp.float32),
            pltpu.VMEM_SHARED((NPAD, D), jnp.float32),
            pltpu.VMEM_SHARED((NPAD,), jnp.float32),
            pltpu.SemaphoreType.DMA((2,)),
            pltpu.SemaphoreType.DMA((2,)),
        ],
    )
    def a2(ents, rels, table, h, sums_out, cnt_out,
           entb, relb, pidb, tvb, selb, rowsb, onesb, zbuf, zvec,
           sums_sh, cnt_sh, semi, semg):
        cid = lax.axis_index("c")
        sid = lax.axis_index("s")
        base = _wid() * PWA
        r0 = sid * RPT

        _zero_fill(zbuf)
        _zero_fill_1d(zvec)
        for m in range(CA // 16):
            onesb[0, pl.ds(m * 16, 16)] = jnp.ones((16,), jnp.float32)

        @pl.loop(0, RPT // ZR)
        def _(jj):
            pltpu.sync_copy(zbuf, sums_sh.at[pl.ds(r0 + jj * ZR, ZR)])

        pltpu.sync_copy(zvec, cnt_sh.at[pl.ds(r0, RPT)])
        plsc.subcore_barrier()

        @pl.loop(0, NCHA)
        def _(j):
            off = base + j * CA
            pltpu.async_copy(ents.at[pl.ds(off, CA)], entb.at[0], semi.at[0])
            pltpu.async_copy(rels.at[pl.ds(off, CA)], relb.at[0], semi.at[1])
            pltpu.make_async_copy(ents.at[pl.ds(off, CA)], entb.at[0], semi.at[0]).wait()
            pltpu.make_async_copy(rels.at[pl.ds(off, CA)], relb.at[0], semi.at[1]).wait()
            for m in range(CA // 16):
                sl = pl.ds(m * 16, 16)
                pidb[0, sl] = entb[0, sl] * R2 + relb[0, sl]
            g1 = pltpu.async_copy(table.at[pidb.at[0]], tvb.at[0], semg.at[0])
            g2 = pltpu.async_copy(h.at[entb.at[0]], rowsb.at[0], semg.at[1])
            g1.wait()
            g2.wait()
            for m in range(CA // 16):
                sl = pl.ds(m * 16, 16)
                val16 = (off + m * 16) + lax.iota(jnp.int32, 16)
                first = tvb[0, sl] == val16
                selb[0, sl] = jnp.where(first, relb[0, sl], DUMMY)
            pltpu.sync_copy(rowsb.at[0], sums_sh.at[selb.at[0]], add=True)
            pltpu.sync_copy(onesb.at[0], cnt_sh.at[selb.at[0]], add=True)

        plsc.subcore_barrier()

        @pl.loop(0, RPT // 128)
        def _(jj):
            pltpu.sync_copy(sums_sh.at[pl.ds(r0 + jj * 128, 128)],
                            sums_out.at[cid, pl.ds(r0 + jj * 128, 128)])

        pltpu.sync_copy(cnt_sh.at[pl.ds(r0, RPT)], cnt_out.at[cid, pl.ds(r0, RPT)])

    return a2


@functools.lru_cache(maxsize=None)
def _make_b(with_deg):
    outs = [pltpu.HBM((NC, NPAD, D), jnp.float32)]
    scratch = [
        pltpu.VMEM((2, CB), jnp.int32),
        pltpu.VMEM((2, CB), jnp.int32),
        pltpu.VMEM((2, CB), jnp.int32),
        pltpu.VMEM((2, CB, D), jnp.float32),
        pltpu.VMEM((2, CB, D), jnp.float32),
        pltpu.VMEM((1, CB), jnp.float32),
        pltpu.VMEM((ZR, D), jnp.float32),
        pltpu.VMEM((RPT,), jnp.float32),
        pltpu.VMEM_SHARED((NPAD, D), jnp.float32),
        pltpu.VMEM_SHARED((NPAD,), jnp.float32),
        pltpu.SemaphoreType.DMA((3,)),
        pltpu.SemaphoreType.DMA((2,)),
    ]
    if with_deg:
        outs.append(pltpu.HBM((NC, NPAD), jnp.float32))

    @functools.partial(
        pl.kernel,
        out_type=tuple(outs),
        mesh=_mesh(),
        scratch_types=scratch,
    )
    def b(src, rel, dst, taba, tabb, *args):
        if with_deg:
            (agg_out, deg_out, sb, rb, db, rowsa, rowsb, onesb, zbuf, zvec,
             agg_sh, deg_sh, semi, semg) = args
        else:
            (agg_out, sb, rb, db, rowsa, rowsb, onesb, zbuf, zvec,
             agg_sh, deg_sh, semi, semg) = args
            deg_out = None
        cid = lax.axis_index("c")
        sid = lax.axis_index("s")
        wid = _wid()
        r0 = sid * RPT

        _zero_fill(zbuf)
        _zero_fill_1d(zvec)
        for m in range(CB // 16):
            onesb[0, pl.ds(m * 16, 16)] = jnp.ones((16,), jnp.float32)

        @pl.loop(0, RPT // ZR)
        def _(jj):
            pltpu.sync_copy(zbuf, agg_sh.at[pl.ds(r0 + jj * ZR, ZR)])

        if with_deg:
            pltpu.sync_copy(zvec, deg_sh.at[pl.ds(r0, RPT)])
        plsc.subcore_barrier()

        @pl.loop(0, NCHB)
        def _(k):
            off = (wid * NCHB + k) * CB
            pltpu.async_copy(src.at[pl.ds(off, CB)], sb.at[0], semi.at[0])
            pltpu.async_copy(rel.at[pl.ds(off, CB)], rb.at[0], semi.at[1])
            pltpu.async_copy(dst.at[pl.ds(off, CB)], db.at[0], semi.at[2])
            pltpu.make_async_copy(src.at[pl.ds(off, CB)], sb.at[0], semi.at[0]).wait()
            pltpu.make_async_copy(rel.at[pl.ds(off, CB)], rb.at[0], semi.at[1]).wait()
            pltpu.make_async_copy(dst.at[pl.ds(off, CB)], db.at[0], semi.at[2]).wait()
            g1 = pltpu.async_copy(taba.at[sb.at[0]], rowsa.at[0], semg.at[0])
            g2 = pltpu.async_copy(tabb.at[rb.at[0]], rowsb.at[0], semg.at[1])
            g1.wait()
            g2.wait()
            pltpu.sync_copy(rowsa.at[0], agg_sh.at[db.at[0]], add=True)
            pltpu.sync_copy(rowsb.at[0], agg_sh.at[db.at[0]], add=True)
            if with_deg:
                pltpu.sync_copy(onesb.at[0], deg_sh.at[db.at[0]], add=True)

        plsc.subcore_barrier()

        @pl.loop(0, RPT // 128)
        def _(jj):
            pltpu.sync_copy(agg_sh.at[pl.ds(r0 + jj * 128, 128)],
                            agg_out.at[cid, pl.ds(r0 + jj * 128, 128)])

        if with_deg:
            pltpu.sync_copy(deg_sh.at[pl.ds(r0, RPT)], deg_out.at[cid, pl.ds(r0, RPT)])

    return b


def _tc_norm(x):
    def body(x_ref, o_ref):
        v = x_ref[...]
        n = jnp.sqrt(jnp.sum(v * v, axis=1, keepdims=True))
        o_ref[...] = v / jnp.maximum(n, 1e-12)

    return pl.pallas_call(
        body,
        out_shape=jax.ShapeDtypeStruct((N, D), jnp.float32),
        grid=(N // TCB,),
        in_specs=[pl.BlockSpec((TCB, D), lambda i: (i, 0))],
        out_specs=pl.BlockSpec((TCB, D), lambda i: (i, 0)),
    )(x)


def _tc_rel(sums2, counts3, srel, hidden, wihs, wihd, whh, bih, bhh):
    def body(s_ref, c_ref, sr_ref, hid_ref, wihs_ref, wihd_ref, whh_ref,
             bih_ref, bhh_ref, o_ref):
        s = s_ref[0] + s_ref[1]
        c = c_ref[0] + c_ref[1]
        dyn = jnp.where(c > 0.0, s / jnp.maximum(c, 1.0), 0.0)
        gi = (jnp.dot(sr_ref[...], wihs_ref[...], preferred_element_type=jnp.float32)
              + jnp.dot(dyn, wihd_ref[...], preferred_element_type=jnp.float32)
              + bih_ref[...])
        gh = (jnp.dot(hid_ref[...], whh_ref[...], preferred_element_type=jnp.float32)
              + bhh_ref[...])
        rg = jax.nn.sigmoid(gi[:, :D] + gh[:, :D])
        zg = jax.nn.sigmoid(gi[:, D:2 * D] + gh[:, D:2 * D])
        ng = jnp.tanh(gi[:, 2 * D:] + rg * gh[:, 2 * D:])
        o_ref[...] = (1.0 - zg) * ng + zg * hid_ref[...]

    return pl.pallas_call(
        body,
        out_shape=jax.ShapeDtypeStruct((R2, D), jnp.float32),
        grid=(R2 // TCB,),
        in_specs=[
            pl.BlockSpec((NC, TCB, D), lambda i: (0, i, 0)),
            pl.BlockSpec((NC, TCB, 1), lambda i: (0, i, 0)),
            pl.BlockSpec((TCB, D), lambda i: (i, 0)),
            pl.BlockSpec((TCB, D), lambda i: (i, 0)),
            pl.BlockSpec((D, 3 * D), lambda i: (0, 0)),
            pl.BlockSpec((D, 3 * D), lambda i: (0, 0)),
            pl.BlockSpec((D, 3 * D), lambda i: (0, 0)),
            pl.BlockSpec((1, 3 * D), lambda i: (0, 0)),
            pl.BlockSpec((1, 3 * D), lambda i: (0, 0)),
        ],
        out_specs=pl.BlockSpec((TCB, D), lambda i: (i, 0)),
    )(sums2, counts3, srel, hidden, wihs, wihd, whh, bih, bhh)


def _tc_layer0(agg2, deg3, cur, nw, lw):
    def body(a_ref, d_ref, cur_ref, nw_ref, lw_ref, o_ref):
        a = a_ref[0] + a_ref[1]
        d = jnp.maximum(d_ref[0] + d_ref[1], 1.0)
        o_ref[...] = (jnp.dot(a, nw_ref[...], preferred_element_type=jnp.float32) / d
                      + jnp.dot(cur_ref[...], lw_ref[...],
                                preferred_element_type=jnp.float32))

    return pl.pallas_call(
        body,
        out_shape=jax.ShapeDtypeStruct((N, D), jnp.float32),
        grid=(N // TCB,),
        in_specs=[
            pl.BlockSpec((NC, TCB, D), lambda i: (0, i, 0)),
            pl.BlockSpec((NC, TCB, 1), lambda i: (0, i, 0)),
            pl.BlockSpec((TCB, D), lambda i: (i, 0)),
            pl.BlockSpec((D, D), lambda i: (0, 0)),
            pl.BlockSpec((D, D), lambda i: (0, 0)),
        ],
        out_specs=pl.BlockSpec((TCB, D), lambda i: (i, 0)),
    )(agg2, deg3, cur, nw, lw)


def _tc_layer1(agg2, deg3, cur, h, nw, lw, gw, gb):
    def body(a_ref, d_ref, cur_ref, h_ref, nw_ref, lw_ref, gw_ref, gb_ref, o_ref):
        a = a_ref[0] + a_ref[1]
        d = jnp.maximum(d_ref[0] + d_ref[1], 1.0)
        cur2 = (jnp.dot(a, nw_ref[...], preferred_element_type=jnp.float32) / d
                + jnp.dot(cur_ref[...], lw_ref[...],
                          preferred_element_type=jnp.float32))
        g = jax.nn.sigmoid(
            jnp.dot(h_ref[...], gw_ref[...], preferred_element_type=jnp.float32)
            + gb_ref[...])
        o_ref[...] = g * cur2 + (1.0 - g) * h_ref[...]

    return pl.pallas_call(
        body,
        out_shape=jax.ShapeDtypeStruct((N, D), jnp.float32),
        grid=(N // TCB,),
        in_specs=[
            pl.BlockSpec((NC, TCB, D), lambda i: (0, i, 0)),
            pl.BlockSpec((NC, TCB, 1), lambda i: (0, i, 0)),
            pl.BlockSpec((TCB, D), lambda i: (i, 0)),
            pl.BlockSpec((TCB, D), lambda i: (i, 0)),
            pl.BlockSpec((D, D), lambda i: (0, 0)),
            pl.BlockSpec((D, D), lambda i: (0, 0)),
            pl.BlockSpec((D, D), lambda i: (0, 0)),
            pl.BlockSpec((1, D), lambda i: (0, 0)),
        ],
        out_specs=pl.BlockSpec((TCB, D), lambda i: (i, 0)),
    )(agg2, deg3, cur, h, nw, lw, gw, gb)


def kernel(edges, static_entity_embed, static_relation_embed, gate_weight,
           gate_bias, gru_w_ih, gru_w_hh, gru_b_ih, gru_b_hh, neigh_w, loop_w):
    et = edges.transpose(0, 2, 1)
    wihs = gru_w_ih[:, :D].T
    wihd = gru_w_ih[:, D:].T
    whh = gru_w_hh.T
    bih = gru_b_ih.reshape(1, 3 * D)
    bhh = gru_b_hh.reshape(1, 3 * D)
    gb = gate_bias.reshape(1, D)

    pad0 = jnp.zeros((EPAD - E,), jnp.int32)
    padd = jnp.full((EPAD - E,), DUMMY, jnp.int32)

    a1 = _make_a1()
    a2 = _make_a2()
    b_deg = _make_b(True)
    b_nodeg = _make_b(False)

    h = _tc_norm(static_entity_embed)
    evolved = static_relation_embed
    outs = []
    for t in range(T):
        src, rel, dst = et[t, 0], et[t, 1], et[t, 2]
        ents = jnp.concatenate([src, dst])
        rels2 = jnp.concatenate([rel, rel])
        src_p = jnp.concatenate([src, pad0])
        rel_p = jnp.concatenate([rel, pad0])
        dst_p = jnp.concatenate([dst, padd])

        table = a1(ents, rels2)
        sums2, counts2 = a2(ents, rels2, table, h)
        evolved = _tc_rel(sums2, counts2.reshape(NC, NPAD, 1),
                          static_relation_embed, evolved,
                          wihs, wihd, whh, bih, bhh)

        agg2, deg2 = b_deg(src_p, rel_p, dst_p, h, evolved)
        deg3 = deg2.reshape(NC, NPAD, 1)
        cur1 = _tc_layer0(agg2, deg3, h, neigh_w[0], loop_w[0])

        (agg2b,) = b_nodeg(src_p, rel_p, dst_p, cur1, evolved)
        h = _tc_layer1(agg2b, deg3, cur1, h, neigh_w[1], loop_w[1],
                       gate_weight, gb)
        outs.append(h)

    return jnp.stack(outs, axis=0), evolved


# speedup vs baseline: 3.6166x; 1.2533x over previous
import functools

import jax
import jax.numpy as jnp
from jax import lax
from jax.experimental import pallas as pl
from jax.experimental.pallas import tpu as pltpu
from jax.experimental.pallas import tpu_sc as plsc

N = 10000
R2 = 10000
D = 128
E = 160000
T = 3
P = 2 * E
TBL = N * R2

NC = 2
NS = 16
NW = NC * NS

NPAD = 10240
DUMMY = 10000
RPT = NPAD // NS

CA = 80
PWA = P // NW
NCHA = PWA // CA

CB = 64
NCHB = 80
EPAD = NW * NCHB * CB

ZR = 64

TCB = 1000


@functools.lru_cache(maxsize=None)
def _mesh():
    return plsc.VectorSubcoreMesh(core_axis_name="c", subcore_axis_name="s")


def _wid():
    return lax.axis_index("c") * NS + lax.axis_index("s")


def _zero_fill(zbuf):
    rows = zbuf.shape[0]

    @pl.loop(0, rows)
    def _(i):
        for k in range(D // 16):
            zbuf[i, pl.ds(k * 16, 16)] = jnp.zeros((16,), jnp.float32)


def _zero_fill_1d(zvec):
    n = zvec.shape[0]

    @pl.loop(0, n // 16)
    def _(i):
        zvec[pl.ds(i * 16, 16)] = jnp.zeros((16,), jnp.float32)


@functools.lru_cache(maxsize=None)
def _make_a1():
    @functools.partial(
        pl.kernel,
        out_type=pltpu.HBM((TBL,), jnp.int32),
        mesh=_mesh(),
        scratch_types=[
            pltpu.VMEM((2, CA), jnp.int32),
            pltpu.VMEM((2, CA), jnp.int32),
            pltpu.VMEM((2, CA), jnp.int32),
            pltpu.VMEM((2, CA), jnp.int32),
            pltpu.SemaphoreType.DMA((2,)),
            pltpu.SemaphoreType.DMA((2,)),
        ],
    )
    def a1(ents, rels, table, entb, relb, pidb, valb, semi, sems):
        base = _wid() * PWA

        def fire_idx(j, b):
            off = base + j * CA
            pltpu.async_copy(ents.at[pl.ds(off, CA)], entb.at[b], semi.at[b])
            pltpu.async_copy(rels.at[pl.ds(off, CA)], relb.at[b], semi.at[b])

        def wait_idx(j, b):
            off = base + j * CA
            pltpu.make_async_copy(ents.at[pl.ds(off, CA)], entb.at[b], semi.at[b]).wait()
            pltpu.make_async_copy(rels.at[pl.ds(off, CA)], relb.at[b], semi.at[b]).wait()

        def wait_scatter(b):
            pltpu.make_async_copy(valb.at[b], table.at[pidb.at[b]], sems.at[b]).wait()

        def step(j, b):
            off = base + j * CA
            wait_idx(j, b)

            @pl.when(j >= 2)
            def _():
                wait_scatter(b)

            for m in range(CA // 16):
                sl = pl.ds(m * 16, 16)
                pidb[b, sl] = entb[b, sl] * R2 + relb[b, sl]
                valb[b, sl] = (off + m * 16) + lax.iota(jnp.int32, 16)
            pltpu.async_copy(valb.at[b], table.at[pidb.at[b]], sems.at[b])

            @pl.when(j + 2 < NCHA)
            def _():
                fire_idx(j + 2, b)

        fire_idx(0, 0)
        fire_idx(1, 1)

        @pl.loop(0, NCHA - 1, step=2)
        def _(j0):
            step(j0, 0)
            step(j0 + 1, 1)

        step(NCHA - 1, 0)
        wait_scatter(1)
        wait_scatter(0)

    return a1


@functools.lru_cache(maxsize=None)
def _make_a2():
    @functools.partial(
        pl.kernel,
        out_type=(
            pltpu.HBM((NC, NPAD, D), jnp.float32),
            pltpu.HBM((NC, NPAD), jnp.float32),
        ),
        mesh=_mesh(),
        scratch_types=[
            pltpu.VMEM((2, CA), jnp.int32),
            pltpu.VMEM((2, CA), jnp.int32),
            pltpu.VMEM((2, CA), jnp.int32),
            pltpu.VMEM((2, CA), jnp.int32),
            pltpu.VMEM((2, CA), jnp.int32),
            pltpu.VMEM((2, CA, D), jnp.float32),
            pltpu.VMEM((1, CA), jnp.float32),
            pltpu.VMEM((ZR, D), jnp.float32),
            pltpu.VMEM((RPT,), jnp.float32),
            pltpu.VMEM_SHARED((NPAD, D), jnp.float32),
            pltpu.VMEM_SHARED((NPAD,), jnp.float32),
            pltpu.SemaphoreType.DMA((2,)),
            pltpu.SemaphoreType.DMA((2,)),
        ],
    )
    def a2(ents, rels, table, h, sums_out, cnt_out,
           entb, relb, pidb, tvb, selb, rowsb, onesb, zbuf, zvec,
           sums_sh, cnt_sh, semi, semg):
        cid = lax.axis_index("c")
        sid = lax.axis_index("s")
        base = _wid() * PWA
        r0 = sid * RPT

        _zero_fill(zbuf)
        _zero_fill_1d(zvec)
        for m in range(CA // 16):
            onesb[0, pl.ds(m * 16, 16)] = jnp.ones((16,), jnp.float32)

        @pl.loop(0, RPT // ZR)
        def _(jj):
            pltpu.sync_copy(zbuf, sums_sh.at[pl.ds(r0 + jj * ZR, ZR)])

        pltpu.sync_copy(zvec, cnt_sh.at[pl.ds(r0, RPT)])
        plsc.subcore_barrier()

        def fire_idx(j, b):
            off = base + j * CA
            pltpu.async_copy(ents.at[pl.ds(off, CA)], entb.at[b], semi.at[b])
            pltpu.async_copy(rels.at[pl.ds(off, CA)], relb.at[b], semi.at[b])

        def fire_gather(j, b):
            off = base + j * CA
            pltpu.make_async_copy(ents.at[pl.ds(off, CA)], entb.at[b], semi.at[b]).wait()
            pltpu.make_async_copy(rels.at[pl.ds(off, CA)], relb.at[b], semi.at[b]).wait()
            for m in range(CA // 16):
                sl = pl.ds(m * 16, 16)
                pidb[b, sl] = entb[b, sl] * R2 + relb[b, sl]
            pltpu.async_copy(table.at[pidb.at[b]], tvb.at[b], semg.at[b])
            pltpu.async_copy(h.at[entb.at[b]], rowsb.at[b], semg.at[b])

        def consume(j, b):
            off = base + j * CA
            pltpu.make_async_copy(table.at[pidb.at[b]], tvb.at[b], semg.at[b]).wait()
            pltpu.make_async_copy(h.at[entb.at[b]], rowsb.at[b], semg.at[b]).wait()
            for m in range(CA // 16):
                sl = pl.ds(m * 16, 16)
                val16 = (off + m * 16) + lax.iota(jnp.int32, 16)
                first = tvb[b, sl] == val16
                selb[b, sl] = jnp.where(first, relb[b, sl], DUMMY)
            pltpu.sync_copy(rowsb.at[b], sums_sh.at[selb.at[b]], add=True)
            pltpu.sync_copy(onesb.at[0], cnt_sh.at[selb.at[b]], add=True)

        fire_idx(0, 0)
        fire_idx(1, 1)
        fire_gather(0, 0)

        @pl.loop(0, NCHA - 1, step=2)
        def _(j0):
            for b in range(2):
                j = j0 + b
                fire_gather(j + 1, 1 - b)
                consume(j, b)

                @pl.when(j + 2 < NCHA)
                def _():
                    fire_idx(j + 2, b)

        consume(NCHA - 1, 0)
        plsc.subcore_barrier()

        @pl.loop(0, RPT // 128)
        def _(jj):
            pltpu.sync_copy(sums_sh.at[pl.ds(r0 + jj * 128, 128)],
                            sums_out.at[cid, pl.ds(r0 + jj * 128, 128)])

        pltpu.sync_copy(cnt_sh.at[pl.ds(r0, RPT)], cnt_out.at[cid, pl.ds(r0, RPT)])

    return a2


@functools.lru_cache(maxsize=None)
def _make_b(with_deg):
    outs = [pltpu.HBM((NC, NPAD, D), jnp.float32)]
    scratch = [
        pltpu.VMEM((2, CB), jnp.int32),
        pltpu.VMEM((2, CB), jnp.int32),
        pltpu.VMEM((2, CB), jnp.int32),
        pltpu.VMEM((2, CB, D), jnp.float32),
        pltpu.VMEM((2, CB, D), jnp.float32),
        pltpu.VMEM((1, CB), jnp.float32),
        pltpu.VMEM((ZR, D), jnp.float32),
        pltpu.VMEM((RPT,), jnp.float32),
        pltpu.VMEM_SHARED((NPAD, D), jnp.float32),
        pltpu.VMEM_SHARED((NPAD,), jnp.float32),
        pltpu.SemaphoreType.DMA((3,)),
        pltpu.SemaphoreType.DMA((2,)),
    ]
    if with_deg:
        outs.append(pltpu.HBM((NC, NPAD), jnp.float32))

    @functools.partial(
        pl.kernel,
        out_type=tuple(outs),
        mesh=_mesh(),
        scratch_types=scratch,
    )
    def b(src, rel, dst, taba, tabb, *args):
        if with_deg:
            (agg_out, deg_out, sb, rb, db, rowsa, rowsb, onesb, zbuf, zvec,
             agg_sh, deg_sh, semi, semg) = args
        else:
            (agg_out, sb, rb, db, rowsa, rowsb, onesb, zbuf, zvec,
             agg_sh, deg_sh, semi, semg) = args
            deg_out = None
        cid = lax.axis_index("c")
        sid = lax.axis_index("s")
        wid = _wid()
        r0 = sid * RPT

        _zero_fill(zbuf)
        _zero_fill_1d(zvec)
        for m in range(CB // 16):
            onesb[0, pl.ds(m * 16, 16)] = jnp.ones((16,), jnp.float32)

        @pl.loop(0, RPT // ZR)
        def _(jj):
            pltpu.sync_copy(zbuf, agg_sh.at[pl.ds(r0 + jj * ZR, ZR)])

        if with_deg:
            pltpu.sync_copy(zvec, deg_sh.at[pl.ds(r0, RPT)])
        plsc.subcore_barrier()

        def fire_idx(j, b):
            off = (wid * NCHB + j) * CB
            pltpu.async_copy(src.at[pl.ds(off, CB)], sb.at[b], semi.at[b])
            pltpu.async_copy(rel.at[pl.ds(off, CB)], rb.at[b], semi.at[b])
            pltpu.async_copy(dst.at[pl.ds(off, CB)], db.at[b], semi.at[b])

        def fire_gather(j, b):
            off = (wid * NCHB + j) * CB
            pltpu.make_async_copy(src.at[pl.ds(off, CB)], sb.at[b], semi.at[b]).wait()
            pltpu.make_async_copy(rel.at[pl.ds(off, CB)], rb.at[b], semi.at[b]).wait()
            pltpu.make_async_copy(dst.at[pl.ds(off, CB)], db.at[b], semi.at[b]).wait()
            pltpu.async_copy(taba.at[sb.at[b]], rowsa.at[b], semg.at[b])
            pltpu.async_copy(tabb.at[rb.at[b]], rowsb.at[b], semg.at[b])

        def consume(j, b):
            pltpu.make_async_copy(taba.at[sb.at[b]], rowsa.at[b], semg.at[b]).wait()
            pltpu.make_async_copy(tabb.at[rb.at[b]], rowsb.at[b], semg.at[b]).wait()
            pltpu.sync_copy(rowsa.at[b], agg_sh.at[db.at[b]], add=True)
            pltpu.sync_copy(rowsb.at[b], agg_sh.at[db.at[b]], add=True)
            if with_deg:
                pltpu.sync_copy(onesb.at[0], deg_sh.at[db.at[b]], add=True)

        fire_idx(0, 0)
        fire_idx(1, 1)
        fire_gather(0, 0)

        @pl.loop(0, NCHB - 2, step=2)
        def _(j0):
            for b in range(2):
                j = j0 + b
                fire_gather(j + 1, 1 - b)
                consume(j, b)

                @pl.when(j + 2 < NCHB)
                def _():
                    fire_idx(j + 2, b)

        fire_gather(NCHB - 1, 1)
        consume(NCHB - 2, 0)
        consume(NCHB - 1, 1)
        plsc.subcore_barrier()

        @pl.loop(0, RPT // 128)
        def _(jj):
            pltpu.sync_copy(agg_sh.at[pl.ds(r0 + jj * 128, 128)],
                            agg_out.at[cid, pl.ds(r0 + jj * 128, 128)])

        if with_deg:
            pltpu.sync_copy(deg_sh.at[pl.ds(r0, RPT)], deg_out.at[cid, pl.ds(r0, RPT)])

    return b


def _tc_norm(x):
    def body(x_ref, o_ref):
        v = x_ref[...]
        n = jnp.sqrt(jnp.sum(v * v, axis=1, keepdims=True))
        o_ref[...] = v / jnp.maximum(n, 1e-12)

    return pl.pallas_call(
        body,
        out_shape=jax.ShapeDtypeStruct((N, D), jnp.float32),
        grid=(N // TCB,),
        in_specs=[pl.BlockSpec((TCB, D), lambda i: (i, 0))],
        out_specs=pl.BlockSpec((TCB, D), lambda i: (i, 0)),
    )(x)


def _tc_rel(sums2, counts3, srel, hidden, wihs, wihd, whh, bih, bhh):
    def body(s_ref, c_ref, sr_ref, hid_ref, wihs_ref, wihd_ref, whh_ref,
             bih_ref, bhh_ref, o_ref):
        s = s_ref[0] + s_ref[1]
        c = c_ref[0] + c_ref[1]
        dyn = jnp.where(c > 0.0, s / jnp.maximum(c, 1.0), 0.0)
        gi = (jnp.dot(sr_ref[...], wihs_ref[...], preferred_element_type=jnp.float32)
              + jnp.dot(dyn, wihd_ref[...], preferred_element_type=jnp.float32)
              + bih_ref[...])
        gh = (jnp.dot(hid_ref[...], whh_ref[...], preferred_element_type=jnp.float32)
              + bhh_ref[...])
        rg = jax.nn.sigmoid(gi[:, :D] + gh[:, :D])
        zg = jax.nn.sigmoid(gi[:, D:2 * D] + gh[:, D:2 * D])
        ng = jnp.tanh(gi[:, 2 * D:] + rg * gh[:, 2 * D:])
        o_ref[...] = (1.0 - zg) * ng + zg * hid_ref[...]

    return pl.pallas_call(
        body,
        out_shape=jax.ShapeDtypeStruct((R2, D), jnp.float32),
        grid=(R2 // TCB,),
        in_specs=[
            pl.BlockSpec((NC, TCB, D), lambda i: (0, i, 0)),
            pl.BlockSpec((NC, TCB, 1), lambda i: (0, i, 0)),
            pl.BlockSpec((TCB, D), lambda i: (i, 0)),
            pl.BlockSpec((TCB, D), lambda i: (i, 0)),
            pl.BlockSpec((D, 3 * D), lambda i: (0, 0)),
            pl.BlockSpec((D, 3 * D), lambda i: (0, 0)),
            pl.BlockSpec((D, 3 * D), lambda i: (0, 0)),
            pl.BlockSpec((1, 3 * D), lambda i: (0, 0)),
            pl.BlockSpec((1, 3 * D), lambda i: (0, 0)),
        ],
        out_specs=pl.BlockSpec((TCB, D), lambda i: (i, 0)),
    )(sums2, counts3, srel, hidden, wihs, wihd, whh, bih, bhh)


def _tc_layer0(agg2, deg3, cur, nw, lw):
    def body(a_ref, d_ref, cur_ref, nw_ref, lw_ref, o_ref):
        a = a_ref[0] + a_ref[1]
        d = jnp.maximum(d_ref[0] + d_ref[1], 1.0)
        o_ref[...] = (jnp.dot(a, nw_ref[...], preferred_element_type=jnp.float32) / d
                      + jnp.dot(cur_ref[...], lw_ref[...],
                                preferred_element_type=jnp.float32))

    return pl.pallas_call(
        body,
        out_shape=jax.ShapeDtypeStruct((N, D), jnp.float32),
        grid=(N // TCB,),
        in_specs=[
            pl.BlockSpec((NC, TCB, D), lambda i: (0, i, 0)),
            pl.BlockSpec((NC, TCB, 1), lambda i: (0, i, 0)),
            pl.BlockSpec((TCB, D), lambda i: (i, 0)),
            pl.BlockSpec((D, D), lambda i: (0, 0)),
            pl.BlockSpec((D, D), lambda i: (0, 0)),
        ],
        out_specs=pl.BlockSpec((TCB, D), lambda i: (i, 0)),
    )(agg2, deg3, cur, nw, lw)


def _tc_layer1(agg2, deg3, cur, h, nw, lw, gw, gb):
    def body(a_ref, d_ref, cur_ref, h_ref, nw_ref, lw_ref, gw_ref, gb_ref, o_ref):
        a = a_ref[0] + a_ref[1]
        d = jnp.maximum(d_ref[0] + d_ref[1], 1.0)
        cur2 = (jnp.dot(a, nw_ref[...], preferred_element_type=jnp.float32) / d
                + jnp.dot(cur_ref[...], lw_ref[...],
                          preferred_element_type=jnp.float32))
        g = jax.nn.sigmoid(
            jnp.dot(h_ref[...], gw_ref[...], preferred_element_type=jnp.float32)
            + gb_ref[...])
        o_ref[...] = g * cur2 + (1.0 - g) * h_ref[...]

    return pl.pallas_call(
        body,
        out_shape=jax.ShapeDtypeStruct((N, D), jnp.float32),
        grid=(N // TCB,),
        in_specs=[
            pl.BlockSpec((NC, TCB, D), lambda i: (0, i, 0)),
            pl.BlockSpec((NC, TCB, 1), lambda i: (0, i, 0)),
            pl.BlockSpec((TCB, D), lambda i: (i, 0)),
            pl.BlockSpec((TCB, D), lambda i: (i, 0)),
            pl.BlockSpec((D, D), lambda i: (0, 0)),
            pl.BlockSpec((D, D), lambda i: (0, 0)),
            pl.BlockSpec((D, D), lambda i: (0, 0)),
            pl.BlockSpec((1, D), lambda i: (0, 0)),
        ],
        out_specs=pl.BlockSpec((TCB, D), lambda i: (i, 0)),
    )(agg2, deg3, cur, h, nw, lw, gw, gb)


def kernel(edges, static_entity_embed, static_relation_embed, gate_weight,
           gate_bias, gru_w_ih, gru_w_hh, gru_b_ih, gru_b_hh, neigh_w, loop_w):
    et = edges.transpose(0, 2, 1)
    wihs = gru_w_ih[:, :D].T
    wihd = gru_w_ih[:, D:].T
    whh = gru_w_hh.T
    bih = gru_b_ih.reshape(1, 3 * D)
    bhh = gru_b_hh.reshape(1, 3 * D)
    gb = gate_bias.reshape(1, D)

    pad0 = jnp.zeros((EPAD - E,), jnp.int32)
    padd = jnp.full((EPAD - E,), DUMMY, jnp.int32)

    a1 = _make_a1()
    a2 = _make_a2()
    b_deg = _make_b(True)
    b_nodeg = _make_b(False)

    h = _tc_norm(static_entity_embed)
    evolved = static_relation_embed
    outs = []
    for t in range(T):
        src, rel, dst = et[t, 0], et[t, 1], et[t, 2]
        ents = jnp.concatenate([src, dst])
        rels2 = jnp.concatenate([rel, rel])
        src_p = jnp.concatenate([src, pad0])
        rel_p = jnp.concatenate([rel, pad0])
        dst_p = jnp.concatenate([dst, padd])

        table = a1(ents, rels2)
        sums2, counts2 = a2(ents, rels2, table, h)
        evolved = _tc_rel(sums2, counts2.reshape(NC, NPAD, 1),
                          static_relation_embed, evolved,
                          wihs, wihd, whh, bih, bhh)

        agg2, deg2 = b_deg(src_p, rel_p, dst_p, h, evolved)
        deg3 = deg2.reshape(NC, NPAD, 1)
        cur1 = _tc_layer0(agg2, deg3, h, neigh_w[0], loop_w[0])

        (agg2b,) = b_nodeg(src_p, rel_p, dst_p, cur1, evolved)
        h = _tc_layer1(agg2b, deg3, cur1, h, neigh_w[1], loop_w[1],
                       gate_weight, gb)
        outs.append(h)

    return jnp.stack(outs, axis=0), evolved
